# baseline (speedup 1.0000x reference)
import functools

import jax
import jax.numpy as jnp
from jax.experimental import pallas as pl
from jax.experimental.pallas import tpu as pltpu
from jax.experimental.pallas import tpu_sc as plsc
from jax import lax


def _make_sc_gather(V, D, B, nchunk=2):
    NW = 32
    b_per_w = B // NW
    b_per_c = b_per_w // nchunk
    mesh = plsc.VectorSubcoreMesh(core_axis_name="c", subcore_axis_name="s")

    @functools.partial(
        pl.kernel, mesh=mesh,
        out_type=jax.ShapeDtypeStruct((B, D), jnp.float32),
        scratch_types=[
            pltpu.VMEM((b_per_c,), jnp.int32),
            pltpu.VMEM((b_per_c, D), jnp.float32),
            pltpu.SemaphoreType.DMA,
        ],
    )
    def k(table_hbm, idx_hbm, out_hbm, idx_v, rows_v, sem):
        wid = lax.axis_index("s") * 2 + lax.axis_index("c")
        for c in range(nchunk):
            base = wid * b_per_w + c * b_per_c
            pltpu.sync_copy(idx_hbm.at[pl.ds(base, b_per_c)], idx_v)
            pltpu.async_copy(table_hbm.at[idx_v], rows_v, sem).wait()
            pltpu.sync_copy(rows_v, out_hbm.at[pl.ds(base, b_per_c)])

    return k


D_MODEL_ = 768
NUM_HEADS_ = 12
HEAD_DIM_ = 64
NUM_EXPERTS_ = 8
TOP_K_ = 2
D_HIDDEN_ = 256

_TILE_T = 512


def _cumsum8(a):
    for k in (1, 2, 4):
        z = jnp.zeros((k, a.shape[1]), a.dtype)
        a = a + jnp.concatenate([z, a[:-k]], axis=0)
    return a


def _fused_body(x_ref, wpin_t_ref, wr_t_ref, win_t_ref, wout_f_ref,
                wpout_t_ref, out_ref):
    f32 = jnp.float32
    xt = x_ref[...]
    xh = jax.lax.dot_general(xt, wpin_t_ref[...], (((1,), (0,)), ((), ())),
                             preferred_element_type=f32)
    head_outs = []
    for h in range(NUM_HEADS_):
        x_h = xh[:, h * HEAD_DIM_:(h + 1) * HEAD_DIM_]
        logits_t = jax.lax.dot_general(wr_t_ref[h], x_h,
                                       (((0,), (1,)), ((), ())),
                                       preferred_element_type=f32)
        m1 = jnp.max(logits_t, axis=0, keepdims=True)
        is_max = (logits_t == m1).astype(f32)
        csum = _cumsum8(is_max)
        mask1 = (is_max > 0.0) & (csum <= 1.0)
        l2 = jnp.where(mask1, -jnp.inf, logits_t)
        m2 = jnp.max(l2, axis=0, keepdims=True)
        is_max2 = (l2 == m2).astype(f32)
        csum2 = _cumsum8(is_max2)
        mask2 = (is_max2 > 0.0) & (csum2 <= 1.0)
        w2 = 1.0 / (1.0 + jnp.exp(m1 - m2))
        w1 = 1.0 - w2
        coef_t = jnp.where(mask1, w1, 0.0) + jnp.where(mask2, w2, 0.0)
        coef = coef_t.T
        hidden = jax.lax.dot_general(x_h.astype(jnp.bfloat16), win_t_ref[h],
                                     (((1,), (0,)), ((), ())),
                                     preferred_element_type=f32)
        hidden = 0.5 * hidden * (1.0 + jax.lax.erf(hidden * 0.7071067811865476))
        hidden = hidden.astype(jnp.bfloat16)
        y_h = None
        for e in range(NUM_EXPERTS_):
            o_e = jax.lax.dot_general(
                hidden[:, e * D_HIDDEN_:(e + 1) * D_HIDDEN_],
                wout_f_ref[h, e * D_HIDDEN_:(e + 1) * D_HIDDEN_, :],
                (((1,), (0,)), ((), ())), preferred_element_type=f32)
            o_e = o_e * coef[:, e:e + 1]
            y_h = o_e if y_h is None else y_h + o_e
        head_outs.append(y_h)
    y = jnp.concatenate(head_outs, axis=1)
    out_ref[...] = jax.lax.dot_general(y.astype(jnp.bfloat16),
                                       wpout_t_ref[...].astype(jnp.bfloat16),
                                       (((1,), (0,)), ((), ())),
                                       preferred_element_type=f32)


@jax.jit
def kernel(x, Wp_in, Wr, Win, Wout, Wp_out):
    B, S, d = x.shape
    T = B * S
    xf = x.reshape(T, d)
    wpin_t = Wp_in.T
    wr_t = Wr.transpose(0, 2, 1)
    win_t = Win.transpose(0, 3, 1, 2).reshape(
        NUM_HEADS_, HEAD_DIM_, NUM_EXPERTS_ * D_HIDDEN_).astype(jnp.bfloat16)
    wout_f = Wout.reshape(
        NUM_HEADS_, NUM_EXPERTS_ * D_HIDDEN_, HEAD_DIM_).astype(jnp.bfloat16)
    wpout_t = Wp_out.T

    grid = (T // _TILE_T,)
    whole = lambda arr: pl.BlockSpec(arr.shape, lambda i: (0,) * arr.ndim)
    out = pl.pallas_call(
        _fused_body,
        grid=grid,
        in_specs=[
            pl.BlockSpec((_TILE_T, d), lambda i: (i, 0)),
            whole(wpin_t),
            whole(wr_t),
            whole(win_t),
            whole(wout_f),
            whole(wpout_t),
        ],
        out_specs=pl.BlockSpec((_TILE_T, d), lambda i: (i, 0)),
        out_shape=jax.ShapeDtypeStruct((T, d), jnp.float32),
    )(xf, wpin_t, wr_t, win_t, wout_f, wpout_t)
    table = jnp.tile(out[:1280, :128], (48, 1))[:61440]
    idx = (jax.lax.iota(jnp.int32, 49152) * 40009) % 61440
    gathered = _make_sc_gather(61440, 128, 49152)(table, idx)
    out = out + gathered[0, 0] * 1e-38
    return out.reshape(B, S, d)


# speedup vs baseline: 1.5676x; 1.5676x over previous
import functools

import jax
import jax.numpy as jnp
from jax.experimental import pallas as pl
from jax.experimental.pallas import tpu as pltpu

D_MODEL_ = 768
NUM_HEADS_ = 12
HEAD_DIM_ = 64
NUM_EXPERTS_ = 8
TOP_K_ = 2
D_HIDDEN_ = 256

_TILE_T = 512


def _cumsum8(a):
    for k in (1, 2, 4):
        z = jnp.zeros((k, a.shape[1]), a.dtype)
        a = a + jnp.concatenate([z, a[:-k]], axis=0)
    return a


def _fused_body(x_ref, wpin_t_ref, wr_t_ref, win_t_ref, wout_f_ref,
                wpout_t_ref, out_ref):
    f32 = jnp.float32
    xt = x_ref[...]
    xh = jax.lax.dot_general(xt, wpin_t_ref[...], (((1,), (0,)), ((), ())),
                             preferred_element_type=f32)
    head_outs = []
    for h in range(NUM_HEADS_):
        x_h = xh[:, h * HEAD_DIM_:(h + 1) * HEAD_DIM_]
        logits_t = jax.lax.dot_general(wr_t_ref[h], x_h,
                                       (((0,), (1,)), ((), ())),
                                       preferred_element_type=f32)
        m1 = jnp.max(logits_t, axis=0, keepdims=True)
        is_max = (logits_t == m1).astype(f32)
        csum = _cumsum8(is_max)
        mask1 = (is_max > 0.0) & (csum <= 1.0)
        l2 = jnp.where(mask1, -jnp.inf, logits_t)
        m2 = jnp.max(l2, axis=0, keepdims=True)
        is_max2 = (l2 == m2).astype(f32)
        csum2 = _cumsum8(is_max2)
        mask2 = (is_max2 > 0.0) & (csum2 <= 1.0)
        w2 = 1.0 / (1.0 + jnp.exp(m1 - m2))
        w1 = 1.0 - w2
        coef_t = jnp.where(mask1, w1, 0.0) + jnp.where(mask2, w2, 0.0)
        coef = coef_t.T
        hidden = jax.lax.dot_general(x_h.astype(jnp.bfloat16), win_t_ref[h],
                                     (((1,), (0,)), ((), ())),
                                     preferred_element_type=f32)
        hidden = 0.5 * hidden * (1.0 + jax.lax.erf(hidden * 0.7071067811865476))
        hidden = hidden.astype(jnp.bfloat16)
        y_h = None
        for e in range(NUM_EXPERTS_):
            o_e = jax.lax.dot_general(
                hidden[:, e * D_HIDDEN_:(e + 1) * D_HIDDEN_],
                wout_f_ref[h, e * D_HIDDEN_:(e + 1) * D_HIDDEN_, :],
                (((1,), (0,)), ((), ())), preferred_element_type=f32)
            o_e = o_e * coef[:, e:e + 1]
            y_h = o_e if y_h is None else y_h + o_e
        head_outs.append(y_h)
    y = jnp.concatenate(head_outs, axis=1)
    out_ref[...] = jax.lax.dot_general(y.astype(jnp.bfloat16),
                                       wpout_t_ref[...].astype(jnp.bfloat16),
                                       (((1,), (0,)), ((), ())),
                                       preferred_element_type=f32)


@jax.jit
def kernel(x, Wp_in, Wr, Win, Wout, Wp_out):
    B, S, d = x.shape
    T = B * S
    xf = x.reshape(T, d)
    wpin_t = Wp_in.T
    wr_t = Wr.transpose(0, 2, 1)
    win_t = Win.transpose(0, 3, 1, 2).reshape(
        NUM_HEADS_, HEAD_DIM_, NUM_EXPERTS_ * D_HIDDEN_).astype(jnp.bfloat16)
    wout_f = Wout.reshape(
        NUM_HEADS_, NUM_EXPERTS_ * D_HIDDEN_, HEAD_DIM_).astype(jnp.bfloat16)
    wpout_t = Wp_out.T

    grid = (T // _TILE_T,)
    whole = lambda arr: pl.BlockSpec(arr.shape, lambda i: (0,) * arr.ndim)
    out = pl.pallas_call(
        _fused_body,
        grid=grid,
        in_specs=[
            pl.BlockSpec((_TILE_T, d), lambda i: (i, 0)),
            whole(wpin_t),
            whole(wr_t),
            whole(win_t),
            whole(wout_f),
            whole(wpout_t),
        ],
        out_specs=pl.BlockSpec((_TILE_T, d), lambda i: (i, 0)),
        out_shape=jax.ShapeDtypeStruct((T, d), jnp.float32),
    )(xf, wpin_t, wr_t, win_t, wout_f, wpout_t)
    return out.reshape(B, S, d)
